# Initial kernel scaffold; baseline (speedup 1.0000x reference)
import functools
import math

import jax
import jax.numpy as jnp
from jax import lax
from jax.experimental import pallas as pl
from jax.experimental.pallas import tpu as pltpu
from jax.experimental.pallas import tpu_sc as plsc

N_NODES = 10000
N_PAD = 10016
N_EDGES = 320000
E_PAD = 327680
CHUNK = 128
N_TILES = 16
CHUNKS_PER_TILE = E_PAD // (N_TILES * CHUNK)
ROWS_PER_TILE_WB = N_NODES // N_TILES
ROWS_PER_TILE_Z = N_PAD // N_TILES
HIDDEN = 256
EMB = 768
G = 128
LOG2 = math.log(2.0)
BLK = 1000
NBLK = N_NODES // BLK


def _edge_agg(hw):
    mesh = plsc.VectorSubcoreMesh(core_axis_name="c", subcore_axis_name="s")
    out_t = (jax.ShapeDtypeStruct((N_NODES, hw), jnp.float32),
             jax.ShapeDtypeStruct((N_NODES, hw), jnp.float32))

    @functools.partial(
        pl.kernel, mesh=mesh, out_type=out_t,
        scratch_types=[
            pltpu.VMEM((CHUNKS_PER_TILE, CHUNK), jnp.int32),
            pltpu.VMEM((CHUNKS_PER_TILE, CHUNK), jnp.int32),
            pltpu.VMEM((CHUNK, hw), jnp.float32),
            pltpu.VMEM_SHARED((N_PAD, hw), jnp.float32),
            pltpu.SemaphoreType.DMA,
        ],
    )
    def k(h_lo, h_hi, src2d, dst2d, zeros, out_lo, out_hi,
          src_v, dst_v, rows_v, acc, sem):
        c = lax.axis_index("c")
        s = lax.axis_index("s")
        pltpu.sync_copy(zeros.at[pl.ds(s * ROWS_PER_TILE_Z, ROWS_PER_TILE_Z)],
                        acc.at[pl.ds(s * ROWS_PER_TILE_Z, ROWS_PER_TILE_Z)])
        pltpu.sync_copy(src2d.at[pl.ds(s * CHUNKS_PER_TILE, CHUNKS_PER_TILE)], src_v)
        pltpu.sync_copy(dst2d.at[pl.ds(s * CHUNKS_PER_TILE, CHUNKS_PER_TILE)], dst_v)
        plsc.subcore_barrier()

        def run(table):
            def body(j, carry):
                pltpu.async_copy(table.at[src_v.at[j]], rows_v, sem).wait()
                pltpu.sync_copy(rows_v, acc.at[dst_v.at[j]], add=True)
                return carry
            lax.fori_loop(0, CHUNKS_PER_TILE, body, 0)

        @pl.when(c == 0)
        def _():
            run(h_lo)

        @pl.when(c == 1)
        def _():
            run(h_hi)

        plsc.subcore_barrier()

        @pl.when(c == 0)
        def _():
            pltpu.sync_copy(acc.at[pl.ds(s * ROWS_PER_TILE_WB, ROWS_PER_TILE_WB)],
                            out_lo.at[pl.ds(s * ROWS_PER_TILE_WB, ROWS_PER_TILE_WB)])

        @pl.when(c == 1)
        def _():
            pltpu.sync_copy(acc.at[pl.ds(s * ROWS_PER_TILE_WB, ROWS_PER_TILE_WB)],
                            out_hi.at[pl.ds(s * ROWS_PER_TILE_WB, ROWS_PER_TILE_WB)])

    return k


def _gin_mlp(d_in):
    hw = d_in // 2

    def body(h_ref, alo_ref, ahi_ref, wa_ref, ba_ref, wb_ref, bb_ref,
             t_ref, cs_ref, cq_ref):
        i = pl.program_id(0)
        agg = jnp.concatenate([alo_ref[...], ahi_ref[...]], axis=1)
        z = h_ref[...] + agg
        t1 = jnp.dot(z, wa_ref[...], preferred_element_type=jnp.float32)
        t1 = jnp.maximum(t1 + ba_ref[...], 0.0)
        t2 = jnp.dot(t1, wb_ref[...], preferred_element_type=jnp.float32)
        t2 = jnp.maximum(t2 + bb_ref[...], 0.0)
        t_ref[...] = t2

        @pl.when(i == 0)
        def _():
            cs_ref[...] = jnp.zeros_like(cs_ref)
            cq_ref[...] = jnp.zeros_like(cq_ref)

        cs_ref[...] += jnp.sum(t2, axis=0, keepdims=True)
        cq_ref[...] += jnp.sum(t2 * t2, axis=0, keepdims=True)

    return pl.pallas_call(
        body, grid=(NBLK,),
        in_specs=[
            pl.BlockSpec((BLK, d_in), lambda i: (i, 0)),
            pl.BlockSpec((BLK, hw), lambda i: (i, 0)),
            pl.BlockSpec((BLK, hw), lambda i: (i, 0)),
            pl.BlockSpec((d_in, HIDDEN), lambda i: (0, 0)),
            pl.BlockSpec((1, HIDDEN), lambda i: (0, 0)),
            pl.BlockSpec((HIDDEN, HIDDEN), lambda i: (0, 0)),
            pl.BlockSpec((1, HIDDEN), lambda i: (0, 0)),
        ],
        out_specs=[
            pl.BlockSpec((BLK, HIDDEN), lambda i: (i, 0)),
            pl.BlockSpec((1, HIDDEN), lambda i: (0, 0)),
            pl.BlockSpec((1, HIDDEN), lambda i: (0, 0)),
        ],
        out_shape=[
            jax.ShapeDtypeStruct((N_NODES, HIDDEN), jnp.float32),
            jax.ShapeDtypeStruct((1, HIDDEN), jnp.float32),
            jax.ShapeDtypeStruct((1, HIDDEN), jnp.float32),
        ],
    )


def _norm_pool():
    def body(t_ref, cs_ref, cq_ref, b_ref, tn_ref, y_ref):
        i = pl.program_id(0)
        mean = cs_ref[...] * (1.0 / N_NODES)
        var = cq_ref[...] * (1.0 / N_NODES) - mean * mean
        inv = lax.rsqrt(var + 1e-5)
        tn = (t_ref[...] - mean) * inv
        tn_ref[...] = tn
        ids = b_ref[...]
        oh = (ids == lax.broadcasted_iota(jnp.int32, (BLK, G), 1))
        oh = oh.astype(jnp.float32)

        @pl.when(i == 0)
        def _():
            y_ref[...] = jnp.zeros_like(y_ref)

        y_ref[...] += lax.dot_general(oh, tn, (((0,), (0,)), ((), ())),
                                      preferred_element_type=jnp.float32)

    return pl.pallas_call(
        body, grid=(NBLK,),
        in_specs=[
            pl.BlockSpec((BLK, HIDDEN), lambda i: (i, 0)),
            pl.BlockSpec((1, HIDDEN), lambda i: (0, 0)),
            pl.BlockSpec((1, HIDDEN), lambda i: (0, 0)),
            pl.BlockSpec((BLK, 1), lambda i: (i, 0)),
        ],
        out_specs=[
            pl.BlockSpec((BLK, HIDDEN), lambda i: (i, 0)),
            pl.BlockSpec((G, HIDDEN), lambda i: (0, 0)),
        ],
        out_shape=[
            jax.ShapeDtypeStruct((N_NODES, HIDDEN), jnp.float32),
            jax.ShapeDtypeStruct((G, HIDDEN), jnp.float32),
        ],
    )


def _ff_global():
    def body(y_ref, w0, b0, w1, b1, w2, b2, ws, bs, g_ref):
        y = y_ref[...]
        h = jnp.maximum(jnp.dot(y, w0[...], preferred_element_type=jnp.float32) + b0[...], 0.0)
        h = jnp.maximum(jnp.dot(h, w1[...], preferred_element_type=jnp.float32) + b1[...], 0.0)
        h = jnp.maximum(jnp.dot(h, w2[...], preferred_element_type=jnp.float32) + b2[...], 0.0)
        g_ref[...] = h + jnp.dot(y, ws[...], preferred_element_type=jnp.float32) + bs[...]

    return pl.pallas_call(
        body,
        out_shape=jax.ShapeDtypeStruct((G, EMB), jnp.float32),
    )


def _ff_local_loss():
    def body(m_ref, w0, b0, w1, b1, w2, b2, ws, bs, g_ref, b2d_ref,
             loss_ref, acc):
        i = pl.program_id(0)

        @pl.when(i == 0)
        def _():
            acc[0] = 0.0
            acc[1] = 0.0

        m = m_ref[...]
        h = jnp.maximum(jnp.dot(m, w0[...], preferred_element_type=jnp.float32) + b0[...], 0.0)
        h = jnp.maximum(jnp.dot(h, w1[...], preferred_element_type=jnp.float32) + b1[...], 0.0)
        h = jnp.maximum(jnp.dot(h, w2[...], preferred_element_type=jnp.float32) + b2[...], 0.0)
        l = h + jnp.dot(m, ws[...], preferred_element_type=jnp.float32) + bs[...]
        res = lax.dot_general(l, g_ref[...], (((1,), (1,)), ((), ())),
                              preferred_element_type=jnp.float32)
        ids = b2d_ref[...]
        pos = (ids == lax.broadcasted_iota(jnp.int32, (BLK, G), 1))
        pos = pos.astype(jnp.float32)

        def sp(z):
            return jnp.maximum(z, 0.0) + jnp.log1p(jnp.exp(-jnp.abs(z)))

        rp = res * pos
        epos = jnp.sum(LOG2 - sp(-rp))
        q = res * (1.0 - pos)
        eneg = jnp.sum(sp(-q) + q - LOG2)
        acc[0] += epos
        acc[1] += eneg

        @pl.when(i == NBLK - 1)
        def _():
            loss_ref[0, 0] = acc[1] / (N_NODES * (G - 1)) - acc[0] / N_NODES

    return pl.pallas_call(
        body, grid=(NBLK,),
        in_specs=[
            pl.BlockSpec((BLK, EMB), lambda i: (i, 0)),
            pl.BlockSpec((EMB, EMB), lambda i: (0, 0)),
            pl.BlockSpec((1, EMB), lambda i: (0, 0)),
            pl.BlockSpec((EMB, EMB), lambda i: (0, 0)),
            pl.BlockSpec((1, EMB), lambda i: (0, 0)),
            pl.BlockSpec((EMB, EMB), lambda i: (0, 0)),
            pl.BlockSpec((1, EMB), lambda i: (0, 0)),
            pl.BlockSpec((EMB, EMB), lambda i: (0, 0)),
            pl.BlockSpec((1, EMB), lambda i: (0, 0)),
            pl.BlockSpec((G, EMB), lambda i: (0, 0)),
            pl.BlockSpec((BLK, 1), lambda i: (i, 0)),
        ],
        out_specs=pl.BlockSpec((1, 1), lambda i: (0, 0)),
        out_shape=jax.ShapeDtypeStruct((1, 1), jnp.float32),
        scratch_shapes=[pltpu.SMEM((2,), jnp.float32)],
    )


def kernel(x, edge_index, batch, num_graphs, params):
    src = edge_index[0]
    dst = edge_index[1]
    pad = E_PAD - N_EDGES
    src2d = jnp.concatenate([src, jnp.zeros((pad,), jnp.int32)]).reshape(-1, CHUNK)
    dst2d = jnp.concatenate(
        [dst, jnp.full((pad,), N_NODES, jnp.int32)]).reshape(-1, CHUNK)
    batch2d = batch.reshape(N_NODES, 1)

    h = x
    xs = []
    ys = []
    for i in range(3):
        d_in = h.shape[1]
        hw = d_in // 2
        zeros = jnp.zeros((N_PAD, hw), jnp.float32)
        a_lo, a_hi = _edge_agg(hw)(h[:, :hw], h[:, hw:], src2d, dst2d, zeros)
        wa = params['gin%d_Wa' % i]
        ba = params['gin%d_ba' % i].reshape(1, HIDDEN)
        wb = params['gin%d_Wb' % i]
        bb = params['gin%d_bb' % i].reshape(1, HIDDEN)
        t_raw, cs, cq = _gin_mlp(d_in)(h, a_lo, a_hi, wa, ba, wb, bb)
        tn, y_i = _norm_pool()(t_raw, cs, cq, batch2d)
        xs.append(tn)
        ys.append(y_i)
        h = tn

    y = jnp.concatenate(ys, axis=1)
    m = jnp.concatenate(xs, axis=1)

    gp = [params['global_W0'], params['global_b0'].reshape(1, EMB),
          params['global_W1'], params['global_b1'].reshape(1, EMB),
          params['global_W2'], params['global_b2'].reshape(1, EMB),
          params['global_Ws'], params['global_bs'].reshape(1, EMB)]
    g_enc = _ff_global()(y, *gp)

    lp = [params['local_W0'], params['local_b0'].reshape(1, EMB),
          params['local_W1'], params['local_b1'].reshape(1, EMB),
          params['local_W2'], params['local_b2'].reshape(1, EMB),
          params['local_Ws'], params['local_bs'].reshape(1, EMB)]
    loss = _ff_local_loss()(m, *lp, g_enc, batch2d)
    return loss[0, 0]


# speedup vs baseline: 2.2107x; 2.2107x over previous
import functools
import math

import jax
import jax.numpy as jnp
from jax import lax
from jax.experimental import pallas as pl
from jax.experimental.pallas import tpu as pltpu
from jax.experimental.pallas import tpu_sc as plsc

N_NODES = 10000
N_PAD = 10112
N_EDGES = 320000
E_PAD = 327680
CHUNK = 128
N_TILES = 16
CHUNKS_PER_TILE = E_PAD // (N_TILES * CHUNK)
WB_BIG = 640
WB_LAST = N_NODES - 15 * WB_BIG
ROWS_PER_TILE_Z = N_PAD // N_TILES
HIDDEN = 256
EMB = 768
G = 128
LOG2 = math.log(2.0)
BLK = 1000
NBLK = N_NODES // BLK


HW = 128


@functools.cache
def _edge_agg():
    mesh = plsc.VectorSubcoreMesh(core_axis_name="c", subcore_axis_name="s")
    out_t = (jax.ShapeDtypeStruct((N_NODES, HW), jnp.float32),
             jax.ShapeDtypeStruct((N_NODES, HW), jnp.float32))
    cpt = CHUNKS_PER_TILE

    @functools.partial(
        pl.kernel, mesh=mesh, out_type=out_t,
        scratch_types=[
            pltpu.VMEM((cpt // 2, CHUNK), jnp.int32),
            pltpu.VMEM((cpt // 2, CHUNK), jnp.int32),
            pltpu.VMEM((CHUNK, HW), jnp.float32),
            pltpu.VMEM_SHARED((N_PAD, HW), jnp.float32),
            pltpu.SemaphoreType.DMA,
        ],
    )
    def k(t_a, t_b, src2, dst2, zeros, out_a, out_b,
          src_v, dst_v, rows_v, acc, sem):
        c = lax.axis_index("c")
        s = lax.axis_index("s")
        pltpu.sync_copy(zeros.at[pl.ds(s * ROWS_PER_TILE_Z, ROWS_PER_TILE_Z)],
                        acc.at[pl.ds(s * ROWS_PER_TILE_Z, ROWS_PER_TILE_Z)])
        ebase = (c * N_TILES + s) * cpt
        plsc.subcore_barrier()

        def run(table):
            for half in range(2):
                pltpu.sync_copy(
                    src2.at[pl.ds(ebase + half * (cpt // 2), cpt // 2)], src_v)
                pltpu.sync_copy(
                    dst2.at[pl.ds(ebase + half * (cpt // 2), cpt // 2)], dst_v)

                def body(j, carry):
                    pltpu.async_copy(table.at[src_v.at[j]], rows_v, sem).wait()
                    pltpu.sync_copy(rows_v, acc.at[dst_v.at[j]], add=True)
                    return carry
                lax.fori_loop(0, cpt // 2, body, 0)

        @pl.when(c == 0)
        def _():
            run(t_a)

        @pl.when(c == 1)
        def _():
            run(t_b)

        plsc.subcore_barrier()

        def wb(out):
            @pl.when(s < 15)
            def _():
                pltpu.sync_copy(acc.at[pl.ds(s * WB_BIG, WB_BIG)],
                                out.at[pl.ds(s * WB_BIG, WB_BIG)])

            @pl.when(s == 15)
            def _():
                pltpu.sync_copy(acc.at[pl.ds(15 * WB_BIG, WB_LAST)],
                                out.at[pl.ds(15 * WB_BIG, WB_LAST)])

        @pl.when(c == 0)
        def _():
            wb(out_a)

        @pl.when(c == 1)
        def _():
            wb(out_b)

    return k


def _gin_mlp(d_in, single_agg):
    n_agg = 1 if single_agg else 2
    aw = d_in if single_agg else d_in // 2

    def body(*refs):
        h_ref = refs[0]
        agg_refs = refs[1:1 + n_agg]
        wa_ref, ba_ref, wb_ref, bb_ref = refs[1 + n_agg:5 + n_agg]
        t_ref, cs_ref, cq_ref = refs[5 + n_agg:]
        i = pl.program_id(0)
        if single_agg:
            agg = agg_refs[0][...]
        else:
            agg = jnp.concatenate([agg_refs[0][...], agg_refs[1][...]], axis=1)
        z = h_ref[...] + agg
        t1 = jnp.dot(z, wa_ref[...], preferred_element_type=jnp.float32)
        t1 = jnp.maximum(t1 + ba_ref[...], 0.0)
        t2 = jnp.dot(t1, wb_ref[...], preferred_element_type=jnp.float32)
        t2 = jnp.maximum(t2 + bb_ref[...], 0.0)
        t_ref[...] = t2

        @pl.when(i == 0)
        def _():
            cs_ref[...] = jnp.zeros_like(cs_ref)
            cq_ref[...] = jnp.zeros_like(cq_ref)

        cs_ref[...] += jnp.sum(t2, axis=0, keepdims=True)
        cq_ref[...] += jnp.sum(t2 * t2, axis=0, keepdims=True)

    return pl.pallas_call(
        body, grid=(NBLK,),
        in_specs=[
            pl.BlockSpec((BLK, d_in), lambda i: (i, 0)),
        ] + [
            pl.BlockSpec((BLK, aw), lambda i: (i, 0))
            for _ in range(n_agg)
        ] + [
            pl.BlockSpec((d_in, HIDDEN), lambda i: (0, 0)),
            pl.BlockSpec((1, HIDDEN), lambda i: (0, 0)),
            pl.BlockSpec((HIDDEN, HIDDEN), lambda i: (0, 0)),
            pl.BlockSpec((1, HIDDEN), lambda i: (0, 0)),
        ],
        out_specs=[
            pl.BlockSpec((BLK, HIDDEN), lambda i: (i, 0)),
            pl.BlockSpec((1, HIDDEN), lambda i: (0, 0)),
            pl.BlockSpec((1, HIDDEN), lambda i: (0, 0)),
        ],
        out_shape=[
            jax.ShapeDtypeStruct((N_NODES, HIDDEN), jnp.float32),
            jax.ShapeDtypeStruct((1, HIDDEN), jnp.float32),
            jax.ShapeDtypeStruct((1, HIDDEN), jnp.float32),
        ],
    )


def _norm_pool():
    def body(t_ref, cs_ref, cq_ref, b_ref, tn_ref, y_ref):
        i = pl.program_id(0)
        mean = cs_ref[...] * (1.0 / N_NODES)
        var = cq_ref[...] * (1.0 / N_NODES) - mean * mean
        inv = lax.rsqrt(var + 1e-5)
        tn = (t_ref[...] - mean) * inv
        tn_ref[...] = tn
        ids = b_ref[...]
        oh = (ids == lax.broadcasted_iota(jnp.int32, (BLK, G), 1))
        oh = oh.astype(jnp.float32)

        @pl.when(i == 0)
        def _():
            y_ref[...] = jnp.zeros_like(y_ref)

        y_ref[...] += lax.dot_general(oh, tn, (((0,), (0,)), ((), ())),
                                      preferred_element_type=jnp.float32)

    return pl.pallas_call(
        body, grid=(NBLK,),
        in_specs=[
            pl.BlockSpec((BLK, HIDDEN), lambda i: (i, 0)),
            pl.BlockSpec((1, HIDDEN), lambda i: (0, 0)),
            pl.BlockSpec((1, HIDDEN), lambda i: (0, 0)),
            pl.BlockSpec((BLK, 1), lambda i: (i, 0)),
        ],
        out_specs=[
            pl.BlockSpec((BLK, HIDDEN), lambda i: (i, 0)),
            pl.BlockSpec((G, HIDDEN), lambda i: (0, 0)),
        ],
        out_shape=[
            jax.ShapeDtypeStruct((N_NODES, HIDDEN), jnp.float32),
            jax.ShapeDtypeStruct((G, HIDDEN), jnp.float32),
        ],
    )


def _ff_global():
    def body(y_ref, w0, b0, w1, b1, w2, b2, ws, bs, g_ref):
        y = y_ref[...]
        h = jnp.maximum(jnp.dot(y, w0[...], preferred_element_type=jnp.float32) + b0[...], 0.0)
        h = jnp.maximum(jnp.dot(h, w1[...], preferred_element_type=jnp.float32) + b1[...], 0.0)
        h = jnp.maximum(jnp.dot(h, w2[...], preferred_element_type=jnp.float32) + b2[...], 0.0)
        g_ref[...] = h + jnp.dot(y, ws[...], preferred_element_type=jnp.float32) + bs[...]

    return pl.pallas_call(
        body,
        out_shape=jax.ShapeDtypeStruct((G, EMB), jnp.float32),
    )


def _ff_local_loss():
    def body(m_ref, w0, b0, w1, b1, w2, b2, ws, bs, g_ref, b2d_ref,
             loss_ref, acc):
        i = pl.program_id(0)

        @pl.when(i == 0)
        def _():
            acc[0] = 0.0
            acc[1] = 0.0

        m = m_ref[...]
        h = jnp.maximum(jnp.dot(m, w0[...], preferred_element_type=jnp.float32) + b0[...], 0.0)
        h = jnp.maximum(jnp.dot(h, w1[...], preferred_element_type=jnp.float32) + b1[...], 0.0)
        h = jnp.maximum(jnp.dot(h, w2[...], preferred_element_type=jnp.float32) + b2[...], 0.0)
        l = h + jnp.dot(m, ws[...], preferred_element_type=jnp.float32) + bs[...]
        res = lax.dot_general(l, g_ref[...], (((1,), (1,)), ((), ())),
                              preferred_element_type=jnp.float32)
        ids = b2d_ref[...]
        pos = (ids == lax.broadcasted_iota(jnp.int32, (BLK, G), 1))
        pos = pos.astype(jnp.float32)

        def sp(z):
            return jnp.maximum(z, 0.0) + jnp.log1p(jnp.exp(-jnp.abs(z)))

        rp = res * pos
        epos = jnp.sum(LOG2 - sp(-rp))
        q = res * (1.0 - pos)
        eneg = jnp.sum(sp(-q) + q - LOG2)
        acc[0] += epos
        acc[1] += eneg

        @pl.when(i == NBLK - 1)
        def _():
            v = acc[1] / (N_NODES * (G - 1)) - acc[0] / N_NODES
            loss_ref[...] = jnp.reshape(v, (1, 1))

    return pl.pallas_call(
        body, grid=(NBLK,),
        in_specs=[
            pl.BlockSpec((BLK, EMB), lambda i: (i, 0)),
            pl.BlockSpec((EMB, EMB), lambda i: (0, 0)),
            pl.BlockSpec((1, EMB), lambda i: (0, 0)),
            pl.BlockSpec((EMB, EMB), lambda i: (0, 0)),
            pl.BlockSpec((1, EMB), lambda i: (0, 0)),
            pl.BlockSpec((EMB, EMB), lambda i: (0, 0)),
            pl.BlockSpec((1, EMB), lambda i: (0, 0)),
            pl.BlockSpec((EMB, EMB), lambda i: (0, 0)),
            pl.BlockSpec((1, EMB), lambda i: (0, 0)),
            pl.BlockSpec((G, EMB), lambda i: (0, 0)),
            pl.BlockSpec((BLK, 1), lambda i: (i, 0)),
        ],
        out_specs=pl.BlockSpec((1, 1), lambda i: (0, 0)),
        out_shape=jax.ShapeDtypeStruct((1, 1), jnp.float32),
        scratch_shapes=[pltpu.SMEM((2,), jnp.float32)],
    )


def kernel(x, edge_index, batch, num_graphs, params):
    src = edge_index[0]
    dst = edge_index[1]
    pad = E_PAD - N_EDGES
    src1 = jnp.concatenate([src, jnp.zeros((pad,), jnp.int32)])
    dst1 = jnp.concatenate([dst, jnp.full((pad,), N_NODES, jnp.int32)])
    src2 = jnp.concatenate([src1, src1]).reshape(-1, CHUNK)
    dst2 = jnp.concatenate([dst1, dst1]).reshape(-1, CHUNK)
    zeros = jnp.zeros((N_PAD, HW), jnp.float32)
    batch2d = batch.reshape(N_NODES, 1)

    h = x
    xs = []
    ys = []
    for i in range(3):
        d_in = h.shape[1]
        single_agg = d_in == HW
        if single_agg:
            a_a, _ = _edge_agg()(h, h, src2, dst2, zeros)
            aggs = [a_a]
        else:
            a_a, a_b = _edge_agg()(h[:, :HW], h[:, HW:], src2, dst2, zeros)
            aggs = [a_a, a_b]
        wa = params['gin%d_Wa' % i]
        ba = params['gin%d_ba' % i].reshape(1, HIDDEN)
        wb = params['gin%d_Wb' % i]
        bb = params['gin%d_bb' % i].reshape(1, HIDDEN)
        t_raw, cs, cq = _gin_mlp(d_in, single_agg)(h, *aggs,
                                                   wa, ba, wb, bb)
        tn, y_i = _norm_pool()(t_raw, cs, cq, batch2d)
        xs.append(tn)
        ys.append(y_i)
        h = tn

    y = jnp.concatenate(ys, axis=1)
    m = jnp.concatenate(xs, axis=1)

    gp = [params['global_W0'], params['global_b0'].reshape(1, EMB),
          params['global_W1'], params['global_b1'].reshape(1, EMB),
          params['global_W2'], params['global_b2'].reshape(1, EMB),
          params['global_Ws'], params['global_bs'].reshape(1, EMB)]
    g_enc = _ff_global()(y, *gp)

    lp = [params['local_W0'], params['local_b0'].reshape(1, EMB),
          params['local_W1'], params['local_b1'].reshape(1, EMB),
          params['local_W2'], params['local_b2'].reshape(1, EMB),
          params['local_Ws'], params['local_bs'].reshape(1, EMB)]
    loss = _ff_local_loss()(m, *lp, g_enc, batch2d)
    return loss[0, 0]


# speedup vs baseline: 2.3118x; 1.0457x over previous
import functools
import math

import jax
import jax.numpy as jnp
from jax import lax
from jax.experimental import pallas as pl
from jax.experimental.pallas import tpu as pltpu
from jax.experimental.pallas import tpu_sc as plsc

N_NODES = 10000
N_PAD = 10112
N_EDGES = 320000
E_PAD = 327680
CHUNK = 128
N_TILES = 16
CHUNKS_PER_TILE = E_PAD // (N_TILES * CHUNK)
WB_BIG = 640
WB_LAST = N_NODES - 15 * WB_BIG
ROWS_PER_TILE_Z = N_PAD // N_TILES
HIDDEN = 256
EMB = 768
G = 128
LOG2 = math.log(2.0)
BLK = 1000
NBLK = N_NODES // BLK


HW = 128


@functools.cache
def _edge_agg():
    mesh = plsc.VectorSubcoreMesh(core_axis_name="c", subcore_axis_name="s")
    out_t = (jax.ShapeDtypeStruct((N_NODES, HW), jnp.float32),
             jax.ShapeDtypeStruct((N_NODES, HW), jnp.float32))
    cpt = CHUNKS_PER_TILE

    @functools.partial(
        pl.kernel, mesh=mesh, out_type=out_t,
        scratch_types=[
            pltpu.VMEM((cpt // 4, CHUNK), jnp.int32),
            pltpu.VMEM((cpt // 4, CHUNK), jnp.int32),
            pltpu.VMEM((CHUNK, HW), jnp.float32),
            pltpu.VMEM((CHUNK, HW), jnp.float32),
            pltpu.VMEM_SHARED((N_PAD, HW), jnp.float32),
            pltpu.SemaphoreType.DMA,
            pltpu.SemaphoreType.DMA,
            pltpu.SemaphoreType.DMA,
            pltpu.SemaphoreType.DMA,
        ],
    )
    def k(t_a, t_b, src2, dst2, zeros, out_a, out_b,
          src_v, dst_v, rows_a, rows_b, acc, sg0, sg1, ss0, ss1):
        c = lax.axis_index("c")
        s = lax.axis_index("s")
        pltpu.sync_copy(zeros.at[pl.ds(s * ROWS_PER_TILE_Z, ROWS_PER_TILE_Z)],
                        acc.at[pl.ds(s * ROWS_PER_TILE_Z, ROWS_PER_TILE_Z)])
        ebase = (c * N_TILES + s) * cpt
        plsc.subcore_barrier()
        q = cpt // 4

        def run(table):
            for stage in range(4):
                pltpu.sync_copy(src2.at[pl.ds(ebase + stage * q, q)], src_v)
                pltpu.sync_copy(dst2.at[pl.ds(ebase + stage * q, q)], dst_v)

                def body(kk, carry):
                    j0 = 2 * kk
                    j1 = 2 * kk + 1
                    ga = pltpu.async_copy(table.at[src_v.at[j0]], rows_a, sg0)
                    gb = pltpu.async_copy(table.at[src_v.at[j1]], rows_b, sg1)
                    ga.wait()
                    sa = pltpu.async_copy(rows_a, acc.at[dst_v.at[j0]], ss0,
                                          add=True)
                    gb.wait()
                    sb = pltpu.async_copy(rows_b, acc.at[dst_v.at[j1]], ss1,
                                          add=True)
                    sa.wait()
                    sb.wait()
                    return carry
                lax.fori_loop(0, q // 2, body, 0)

        @pl.when(c == 0)
        def _():
            run(t_a)

        @pl.when(c == 1)
        def _():
            run(t_b)

        plsc.subcore_barrier()

        def wb(out):
            @pl.when(s < 15)
            def _():
                pltpu.sync_copy(acc.at[pl.ds(s * WB_BIG, WB_BIG)],
                                out.at[pl.ds(s * WB_BIG, WB_BIG)])

            @pl.when(s == 15)
            def _():
                pltpu.sync_copy(acc.at[pl.ds(15 * WB_BIG, WB_LAST)],
                                out.at[pl.ds(15 * WB_BIG, WB_LAST)])

        @pl.when(c == 0)
        def _():
            wb(out_a)

        @pl.when(c == 1)
        def _():
            wb(out_b)

    return k


def _gin_mlp(d_in, single_agg):
    n_agg = 1 if single_agg else 2
    aw = d_in if single_agg else d_in // 2

    def body(*refs):
        h_ref = refs[0]
        agg_refs = refs[1:1 + n_agg]
        wa_ref, ba_ref, wb_ref, bb_ref = refs[1 + n_agg:5 + n_agg]
        t_ref, cs_ref, cq_ref = refs[5 + n_agg:]
        i = pl.program_id(0)
        if single_agg:
            agg = agg_refs[0][...]
        else:
            agg = jnp.concatenate([agg_refs[0][...], agg_refs[1][...]], axis=1)
        z = h_ref[...] + agg
        t1 = jnp.dot(z, wa_ref[...], preferred_element_type=jnp.float32)
        t1 = jnp.maximum(t1 + ba_ref[...], 0.0)
        t2 = jnp.dot(t1, wb_ref[...], preferred_element_type=jnp.float32)
        t2 = jnp.maximum(t2 + bb_ref[...], 0.0)
        t_ref[...] = t2

        @pl.when(i == 0)
        def _():
            cs_ref[...] = jnp.zeros_like(cs_ref)
            cq_ref[...] = jnp.zeros_like(cq_ref)

        cs_ref[...] += jnp.sum(t2, axis=0, keepdims=True)
        cq_ref[...] += jnp.sum(t2 * t2, axis=0, keepdims=True)

    return pl.pallas_call(
        body, grid=(NBLK,),
        in_specs=[
            pl.BlockSpec((BLK, d_in), lambda i: (i, 0)),
        ] + [
            pl.BlockSpec((BLK, aw), lambda i: (i, 0))
            for _ in range(n_agg)
        ] + [
            pl.BlockSpec((d_in, HIDDEN), lambda i: (0, 0)),
            pl.BlockSpec((1, HIDDEN), lambda i: (0, 0)),
            pl.BlockSpec((HIDDEN, HIDDEN), lambda i: (0, 0)),
            pl.BlockSpec((1, HIDDEN), lambda i: (0, 0)),
        ],
        out_specs=[
            pl.BlockSpec((BLK, HIDDEN), lambda i: (i, 0)),
            pl.BlockSpec((1, HIDDEN), lambda i: (0, 0)),
            pl.BlockSpec((1, HIDDEN), lambda i: (0, 0)),
        ],
        out_shape=[
            jax.ShapeDtypeStruct((N_NODES, HIDDEN), jnp.float32),
            jax.ShapeDtypeStruct((1, HIDDEN), jnp.float32),
            jax.ShapeDtypeStruct((1, HIDDEN), jnp.float32),
        ],
    )


def _norm_pool():
    def body(t_ref, cs_ref, cq_ref, b_ref, tn_ref, y_ref):
        i = pl.program_id(0)
        mean = cs_ref[...] * (1.0 / N_NODES)
        var = cq_ref[...] * (1.0 / N_NODES) - mean * mean
        inv = lax.rsqrt(var + 1e-5)
        tn = (t_ref[...] - mean) * inv
        tn_ref[...] = tn
        ids = b_ref[...]
        oh = (ids == lax.broadcasted_iota(jnp.int32, (BLK, G), 1))
        oh = oh.astype(jnp.float32)

        @pl.when(i == 0)
        def _():
            y_ref[...] = jnp.zeros_like(y_ref)

        y_ref[...] += lax.dot_general(oh, tn, (((0,), (0,)), ((), ())),
                                      preferred_element_type=jnp.float32)

    return pl.pallas_call(
        body, grid=(NBLK,),
        in_specs=[
            pl.BlockSpec((BLK, HIDDEN), lambda i: (i, 0)),
            pl.BlockSpec((1, HIDDEN), lambda i: (0, 0)),
            pl.BlockSpec((1, HIDDEN), lambda i: (0, 0)),
            pl.BlockSpec((BLK, 1), lambda i: (i, 0)),
        ],
        out_specs=[
            pl.BlockSpec((BLK, HIDDEN), lambda i: (i, 0)),
            pl.BlockSpec((G, HIDDEN), lambda i: (0, 0)),
        ],
        out_shape=[
            jax.ShapeDtypeStruct((N_NODES, HIDDEN), jnp.float32),
            jax.ShapeDtypeStruct((G, HIDDEN), jnp.float32),
        ],
    )


def _ff_global():
    def body(y_ref, w0, b0, w1, b1, w2, b2, ws, bs, g_ref):
        y = y_ref[...]
        h = jnp.maximum(jnp.dot(y, w0[...], preferred_element_type=jnp.float32) + b0[...], 0.0)
        h = jnp.maximum(jnp.dot(h, w1[...], preferred_element_type=jnp.float32) + b1[...], 0.0)
        h = jnp.maximum(jnp.dot(h, w2[...], preferred_element_type=jnp.float32) + b2[...], 0.0)
        g_ref[...] = h + jnp.dot(y, ws[...], preferred_element_type=jnp.float32) + bs[...]

    return pl.pallas_call(
        body,
        out_shape=jax.ShapeDtypeStruct((G, EMB), jnp.float32),
    )


def _ff_local_loss():
    def body(m_ref, w0, b0, w1, b1, w2, b2, ws, bs, g_ref, b2d_ref,
             loss_ref, acc):
        i = pl.program_id(0)

        @pl.when(i == 0)
        def _():
            acc[0] = 0.0
            acc[1] = 0.0

        m = m_ref[...]
        h = jnp.maximum(jnp.dot(m, w0[...], preferred_element_type=jnp.float32) + b0[...], 0.0)
        h = jnp.maximum(jnp.dot(h, w1[...], preferred_element_type=jnp.float32) + b1[...], 0.0)
        h = jnp.maximum(jnp.dot(h, w2[...], preferred_element_type=jnp.float32) + b2[...], 0.0)
        l = h + jnp.dot(m, ws[...], preferred_element_type=jnp.float32) + bs[...]
        res = lax.dot_general(l, g_ref[...], (((1,), (1,)), ((), ())),
                              preferred_element_type=jnp.float32)
        ids = b2d_ref[...]
        pos = (ids == lax.broadcasted_iota(jnp.int32, (BLK, G), 1))
        pos = pos.astype(jnp.float32)

        def sp(z):
            return jnp.maximum(z, 0.0) + jnp.log1p(jnp.exp(-jnp.abs(z)))

        rp = res * pos
        epos = jnp.sum(LOG2 - sp(-rp))
        q = res * (1.0 - pos)
        eneg = jnp.sum(sp(-q) + q - LOG2)
        acc[0] += epos
        acc[1] += eneg

        @pl.when(i == NBLK - 1)
        def _():
            v = acc[1] / (N_NODES * (G - 1)) - acc[0] / N_NODES
            loss_ref[...] = jnp.reshape(v, (1, 1))

    return pl.pallas_call(
        body, grid=(NBLK,),
        in_specs=[
            pl.BlockSpec((BLK, EMB), lambda i: (i, 0)),
            pl.BlockSpec((EMB, EMB), lambda i: (0, 0)),
            pl.BlockSpec((1, EMB), lambda i: (0, 0)),
            pl.BlockSpec((EMB, EMB), lambda i: (0, 0)),
            pl.BlockSpec((1, EMB), lambda i: (0, 0)),
            pl.BlockSpec((EMB, EMB), lambda i: (0, 0)),
            pl.BlockSpec((1, EMB), lambda i: (0, 0)),
            pl.BlockSpec((EMB, EMB), lambda i: (0, 0)),
            pl.BlockSpec((1, EMB), lambda i: (0, 0)),
            pl.BlockSpec((G, EMB), lambda i: (0, 0)),
            pl.BlockSpec((BLK, 1), lambda i: (i, 0)),
        ],
        out_specs=pl.BlockSpec((1, 1), lambda i: (0, 0)),
        out_shape=jax.ShapeDtypeStruct((1, 1), jnp.float32),
        scratch_shapes=[pltpu.SMEM((2,), jnp.float32)],
    )


def kernel(x, edge_index, batch, num_graphs, params):
    src = edge_index[0]
    dst = edge_index[1]
    pad = E_PAD - N_EDGES
    src1 = jnp.concatenate([src, jnp.zeros((pad,), jnp.int32)])
    dst1 = jnp.concatenate([dst, jnp.full((pad,), N_NODES, jnp.int32)])
    src2 = jnp.concatenate([src1, src1]).reshape(-1, CHUNK)
    dst2 = jnp.concatenate([dst1, dst1]).reshape(-1, CHUNK)
    zeros = jnp.zeros((N_PAD, HW), jnp.float32)
    batch2d = batch.reshape(N_NODES, 1)

    h = x
    xs = []
    ys = []
    for i in range(3):
        d_in = h.shape[1]
        single_agg = d_in == HW
        if single_agg:
            a_a, _ = _edge_agg()(h, h, src2, dst2, zeros)
            aggs = [a_a]
        else:
            a_a, a_b = _edge_agg()(h[:, :HW], h[:, HW:], src2, dst2, zeros)
            aggs = [a_a, a_b]
        wa = params['gin%d_Wa' % i]
        ba = params['gin%d_ba' % i].reshape(1, HIDDEN)
        wb = params['gin%d_Wb' % i]
        bb = params['gin%d_bb' % i].reshape(1, HIDDEN)
        t_raw, cs, cq = _gin_mlp(d_in, single_agg)(h, *aggs,
                                                   wa, ba, wb, bb)
        tn, y_i = _norm_pool()(t_raw, cs, cq, batch2d)
        xs.append(tn)
        ys.append(y_i)
        h = tn

    y = jnp.concatenate(ys, axis=1)
    m = jnp.concatenate(xs, axis=1)

    gp = [params['global_W0'], params['global_b0'].reshape(1, EMB),
          params['global_W1'], params['global_b1'].reshape(1, EMB),
          params['global_W2'], params['global_b2'].reshape(1, EMB),
          params['global_Ws'], params['global_bs'].reshape(1, EMB)]
    g_enc = _ff_global()(y, *gp)

    lp = [params['local_W0'], params['local_b0'].reshape(1, EMB),
          params['local_W1'], params['local_b1'].reshape(1, EMB),
          params['local_W2'], params['local_b2'].reshape(1, EMB),
          params['local_Ws'], params['local_bs'].reshape(1, EMB)]
    loss = _ff_local_loss()(m, *lp, g_enc, batch2d)
    return loss[0, 0]


# speedup vs baseline: 2.6591x; 1.1502x over previous
import functools
import math

import jax
import jax.numpy as jnp
from jax import lax
from jax.experimental import pallas as pl
from jax.experimental.pallas import tpu as pltpu
from jax.experimental.pallas import tpu_sc as plsc

N_NODES = 10000
N_PAD = 10112
N_EDGES = 320000
E_PAD = 327680
CHUNK = 128
N_TILES = 16
CHUNKS_PER_TILE = E_PAD // (N_TILES * CHUNK)
WB_BIG = 640
WB_LAST = N_NODES - 15 * WB_BIG
ROWS_PER_TILE_Z = N_PAD // N_TILES
HIDDEN = 256
EMB = 768
G = 128
LOG2 = math.log(2.0)
BLK = 1000
NBLK = N_NODES // BLK


HW = 128


@functools.cache
def _edge_agg():
    mesh = plsc.VectorSubcoreMesh(core_axis_name="c", subcore_axis_name="s")
    out_t = (jax.ShapeDtypeStruct((N_NODES, HW), jnp.float32),
             jax.ShapeDtypeStruct((N_NODES, HW), jnp.float32))
    cpt = CHUNKS_PER_TILE

    @functools.partial(
        pl.kernel, mesh=mesh, out_type=out_t,
        scratch_types=[
            pltpu.VMEM((cpt // 4, CHUNK), jnp.int32),
            pltpu.VMEM((cpt // 4, CHUNK), jnp.int32),
            pltpu.VMEM((CHUNK, HW), jnp.float32),
            pltpu.VMEM((CHUNK, HW), jnp.float32),
            pltpu.VMEM((16,), jnp.int32),
            pltpu.VMEM_SHARED((N_PAD, HW), jnp.float32),
            pltpu.SemaphoreType.DMA,
            pltpu.SemaphoreType.DMA,
            pltpu.SemaphoreType.DMA,
            pltpu.SemaphoreType.DMA,
        ],
    )
    def k(t_a, t_b, src2, dst2, zeros, cfg, out_a, out_b,
          src_v, dst_v, rows_a, rows_b, cfg_v, acc, sg0, sg1, ss0, ss1):
        c = lax.axis_index("c")
        s = lax.axis_index("s")
        pltpu.sync_copy(zeros.at[pl.ds(s * ROWS_PER_TILE_Z, ROWS_PER_TILE_Z)],
                        acc.at[pl.ds(s * ROWS_PER_TILE_Z, ROWS_PER_TILE_Z)])
        pltpu.sync_copy(cfg, cfg_v)
        cfgv = cfg_v[...]
        n_stages = cfgv[0]
        core_stride = cfgv[1]
        q = cpt // 4
        ebase = c * core_stride + s * (n_stages * q)
        plsc.subcore_barrier()

        def run(table):
            def stage_body(st, carry):
                off = pl.multiple_of(ebase + st * q, 8)
                pltpu.sync_copy(src2.at[pl.ds(off, q)], src_v)
                pltpu.sync_copy(dst2.at[pl.ds(off, q)], dst_v)

                def body(kk, carry2):
                    j0 = 2 * kk
                    j1 = 2 * kk + 1
                    ga = pltpu.async_copy(table.at[src_v.at[j0]], rows_a, sg0)
                    gb = pltpu.async_copy(table.at[src_v.at[j1]], rows_b, sg1)
                    ga.wait()
                    sa = pltpu.async_copy(rows_a, acc.at[dst_v.at[j0]], ss0,
                                          add=True)
                    gb.wait()
                    sb = pltpu.async_copy(rows_b, acc.at[dst_v.at[j1]], ss1,
                                          add=True)
                    sa.wait()
                    sb.wait()
                    return carry2
                lax.fori_loop(0, q // 2, body, 0)
                return carry
            lax.fori_loop(0, n_stages, stage_body, 0)

        @pl.when(c == 0)
        def _():
            run(t_a)

        @pl.when(c == 1)
        def _():
            run(t_b)

        plsc.subcore_barrier()

        def wb(out):
            @pl.when(s < 15)
            def _():
                pltpu.sync_copy(acc.at[pl.ds(s * WB_BIG, WB_BIG)],
                                out.at[pl.ds(s * WB_BIG, WB_BIG)])

            @pl.when(s == 15)
            def _():
                pltpu.sync_copy(acc.at[pl.ds(15 * WB_BIG, WB_LAST)],
                                out.at[pl.ds(15 * WB_BIG, WB_LAST)])

        @pl.when(c == 0)
        def _():
            wb(out_a)

        @pl.when(c == 1)
        def _():
            wb(out_b)

    return k


def _gin_mlp(d_in, sum_parts):
    n_agg = 2
    aw = d_in if sum_parts else d_in // 2

    def body(*refs):
        h_ref = refs[0]
        agg_refs = refs[1:1 + n_agg]
        wa_ref, ba_ref, wb_ref, bb_ref = refs[1 + n_agg:5 + n_agg]
        t_ref, cs_ref, cq_ref = refs[5 + n_agg:]
        i = pl.program_id(0)
        if sum_parts:
            agg = agg_refs[0][...] + agg_refs[1][...]
        else:
            agg = jnp.concatenate([agg_refs[0][...], agg_refs[1][...]], axis=1)
        z = h_ref[...] + agg
        t1 = jnp.dot(z, wa_ref[...], preferred_element_type=jnp.float32)
        t1 = jnp.maximum(t1 + ba_ref[...], 0.0)
        t2 = jnp.dot(t1, wb_ref[...], preferred_element_type=jnp.float32)
        t2 = jnp.maximum(t2 + bb_ref[...], 0.0)
        t_ref[...] = t2

        @pl.when(i == 0)
        def _():
            cs_ref[...] = jnp.zeros_like(cs_ref)
            cq_ref[...] = jnp.zeros_like(cq_ref)

        cs_ref[...] += jnp.sum(t2, axis=0, keepdims=True)
        cq_ref[...] += jnp.sum(t2 * t2, axis=0, keepdims=True)

    return pl.pallas_call(
        body, grid=(NBLK,),
        in_specs=[
            pl.BlockSpec((BLK, d_in), lambda i: (i, 0)),
        ] + [
            pl.BlockSpec((BLK, aw), lambda i: (i, 0))
            for _ in range(n_agg)
        ] + [
            pl.BlockSpec((d_in, HIDDEN), lambda i: (0, 0)),
            pl.BlockSpec((1, HIDDEN), lambda i: (0, 0)),
            pl.BlockSpec((HIDDEN, HIDDEN), lambda i: (0, 0)),
            pl.BlockSpec((1, HIDDEN), lambda i: (0, 0)),
        ],
        out_specs=[
            pl.BlockSpec((BLK, HIDDEN), lambda i: (i, 0)),
            pl.BlockSpec((1, HIDDEN), lambda i: (0, 0)),
            pl.BlockSpec((1, HIDDEN), lambda i: (0, 0)),
        ],
        out_shape=[
            jax.ShapeDtypeStruct((N_NODES, HIDDEN), jnp.float32),
            jax.ShapeDtypeStruct((1, HIDDEN), jnp.float32),
            jax.ShapeDtypeStruct((1, HIDDEN), jnp.float32),
        ],
    )


def _norm_pool():
    def body(t_ref, cs_ref, cq_ref, b_ref, tn_ref, y_ref):
        i = pl.program_id(0)
        mean = cs_ref[...] * (1.0 / N_NODES)
        var = cq_ref[...] * (1.0 / N_NODES) - mean * mean
        inv = lax.rsqrt(var + 1e-5)
        tn = (t_ref[...] - mean) * inv
        tn_ref[...] = tn
        ids = b_ref[...]
        oh = (ids == lax.broadcasted_iota(jnp.int32, (BLK, G), 1))
        oh = oh.astype(jnp.float32)

        @pl.when(i == 0)
        def _():
            y_ref[...] = jnp.zeros_like(y_ref)

        y_ref[...] += lax.dot_general(oh, tn, (((0,), (0,)), ((), ())),
                                      preferred_element_type=jnp.float32)

    return pl.pallas_call(
        body, grid=(NBLK,),
        in_specs=[
            pl.BlockSpec((BLK, HIDDEN), lambda i: (i, 0)),
            pl.BlockSpec((1, HIDDEN), lambda i: (0, 0)),
            pl.BlockSpec((1, HIDDEN), lambda i: (0, 0)),
            pl.BlockSpec((BLK, 1), lambda i: (i, 0)),
        ],
        out_specs=[
            pl.BlockSpec((BLK, HIDDEN), lambda i: (i, 0)),
            pl.BlockSpec((G, HIDDEN), lambda i: (0, 0)),
        ],
        out_shape=[
            jax.ShapeDtypeStruct((N_NODES, HIDDEN), jnp.float32),
            jax.ShapeDtypeStruct((G, HIDDEN), jnp.float32),
        ],
    )


def _ff_global():
    def body(y_ref, w0, b0, w1, b1, w2, b2, ws, bs, g_ref):
        y = y_ref[...]
        h = jnp.maximum(jnp.dot(y, w0[...], preferred_element_type=jnp.float32) + b0[...], 0.0)
        h = jnp.maximum(jnp.dot(h, w1[...], preferred_element_type=jnp.float32) + b1[...], 0.0)
        h = jnp.maximum(jnp.dot(h, w2[...], preferred_element_type=jnp.float32) + b2[...], 0.0)
        g_ref[...] = h + jnp.dot(y, ws[...], preferred_element_type=jnp.float32) + bs[...]

    return pl.pallas_call(
        body,
        out_shape=jax.ShapeDtypeStruct((G, EMB), jnp.float32),
    )


def _ff_local_loss():
    def body(m_ref, w0, b0, w1, b1, w2, b2, ws, bs, g_ref, b2d_ref,
             loss_ref, acc):
        i = pl.program_id(0)

        @pl.when(i == 0)
        def _():
            acc[0] = 0.0
            acc[1] = 0.0

        m = m_ref[...]
        h = jnp.maximum(jnp.dot(m, w0[...], preferred_element_type=jnp.float32) + b0[...], 0.0)
        h = jnp.maximum(jnp.dot(h, w1[...], preferred_element_type=jnp.float32) + b1[...], 0.0)
        h = jnp.maximum(jnp.dot(h, w2[...], preferred_element_type=jnp.float32) + b2[...], 0.0)
        l = h + jnp.dot(m, ws[...], preferred_element_type=jnp.float32) + bs[...]
        res = lax.dot_general(l, g_ref[...], (((1,), (1,)), ((), ())),
                              preferred_element_type=jnp.float32)
        ids = b2d_ref[...]
        pos = (ids == lax.broadcasted_iota(jnp.int32, (BLK, G), 1))
        pos = pos.astype(jnp.float32)

        def sp(z):
            return jnp.maximum(z, 0.0) + jnp.log1p(jnp.exp(-jnp.abs(z)))

        rp = res * pos
        epos = jnp.sum(LOG2 - sp(-rp))
        q = res * (1.0 - pos)
        eneg = jnp.sum(sp(-q) + q - LOG2)
        acc[0] += epos
        acc[1] += eneg

        @pl.when(i == NBLK - 1)
        def _():
            v = acc[1] / (N_NODES * (G - 1)) - acc[0] / N_NODES
            loss_ref[...] = jnp.reshape(v, (1, 1))

    return pl.pallas_call(
        body, grid=(NBLK,),
        in_specs=[
            pl.BlockSpec((BLK, EMB), lambda i: (i, 0)),
            pl.BlockSpec((EMB, EMB), lambda i: (0, 0)),
            pl.BlockSpec((1, EMB), lambda i: (0, 0)),
            pl.BlockSpec((EMB, EMB), lambda i: (0, 0)),
            pl.BlockSpec((1, EMB), lambda i: (0, 0)),
            pl.BlockSpec((EMB, EMB), lambda i: (0, 0)),
            pl.BlockSpec((1, EMB), lambda i: (0, 0)),
            pl.BlockSpec((EMB, EMB), lambda i: (0, 0)),
            pl.BlockSpec((1, EMB), lambda i: (0, 0)),
            pl.BlockSpec((G, EMB), lambda i: (0, 0)),
            pl.BlockSpec((BLK, 1), lambda i: (i, 0)),
        ],
        out_specs=pl.BlockSpec((1, 1), lambda i: (0, 0)),
        out_shape=jax.ShapeDtypeStruct((1, 1), jnp.float32),
        scratch_shapes=[pltpu.SMEM((2,), jnp.float32)],
    )


def kernel(x, edge_index, batch, num_graphs, params):
    src = edge_index[0]
    dst = edge_index[1]
    pad = E_PAD - N_EDGES
    src1 = jnp.concatenate([src, jnp.zeros((pad,), jnp.int32)])
    dst1 = jnp.concatenate([dst, jnp.full((pad,), N_NODES, jnp.int32)])
    src2 = jnp.concatenate([src1, src1]).reshape(-1, CHUNK)
    dst2 = jnp.concatenate([dst1, dst1]).reshape(-1, CHUNK)
    zeros = jnp.zeros((N_PAD, HW), jnp.float32)
    cfg_half = jnp.zeros((16,), jnp.int32).at[0].set(2).at[1].set(1280)
    cfg_full = jnp.zeros((16,), jnp.int32).at[0].set(4).at[1].set(2560)
    batch2d = batch.reshape(N_NODES, 1)

    h = x
    xs = []
    ys = []
    for i in range(3):
        d_in = h.shape[1]
        sum_parts = d_in == HW
        if sum_parts:
            a_a, a_b = _edge_agg()(h, h, src2, dst2, zeros, cfg_half)
        else:
            a_a, a_b = _edge_agg()(h[:, :HW], h[:, HW:], src2, dst2, zeros,
                                   cfg_full)
        wa = params['gin%d_Wa' % i]
        ba = params['gin%d_ba' % i].reshape(1, HIDDEN)
        wb = params['gin%d_Wb' % i]
        bb = params['gin%d_bb' % i].reshape(1, HIDDEN)
        t_raw, cs, cq = _gin_mlp(d_in, sum_parts)(h, a_a, a_b,
                                                  wa, ba, wb, bb)
        tn, y_i = _norm_pool()(t_raw, cs, cq, batch2d)
        xs.append(tn)
        ys.append(y_i)
        h = tn

    y = jnp.concatenate(ys, axis=1)
    m = jnp.concatenate(xs, axis=1)

    gp = [params['global_W0'], params['global_b0'].reshape(1, EMB),
          params['global_W1'], params['global_b1'].reshape(1, EMB),
          params['global_W2'], params['global_b2'].reshape(1, EMB),
          params['global_Ws'], params['global_bs'].reshape(1, EMB)]
    g_enc = _ff_global()(y, *gp)

    lp = [params['local_W0'], params['local_b0'].reshape(1, EMB),
          params['local_W1'], params['local_b1'].reshape(1, EMB),
          params['local_W2'], params['local_b2'].reshape(1, EMB),
          params['local_Ws'], params['local_bs'].reshape(1, EMB)]
    loss = _ff_local_loss()(m, *lp, g_enc, batch2d)
    return loss[0, 0]
